# baseline (speedup 1.0000x reference)
import functools
import math

import jax
import jax.numpy as jnp
from jax import lax
from jax.experimental import pallas as pl
from jax.experimental.pallas import tpu as pltpu
from jax.experimental.pallas import tpu_sc as plsc

_D = 768
_NB = 32
_CR = 16
_NG = _D // 16
_NEG2LOG = -2.0 * math.log(10000.0) / _D


def _tables_body(pq_ref, bb_ref):
    na = pq_ref.shape[1]
    d = lax.broadcasted_iota(jnp.int32, (na, _D), 1)
    inv_freq = jnp.exp((d // 2).astype(jnp.float32) * _NEG2LOG)
    a = lax.broadcasted_iota(jnp.int32, (na, _D), 0).astype(jnp.float32)
    big_ang = (a * float(_NB)) * inv_freq
    pq_ref[0] = jnp.sin(big_ang)
    pq_ref[1] = jnp.sin(big_ang + math.pi / 2.0)

    nb = bb_ref.shape[1]
    db = lax.broadcasted_iota(jnp.int32, (nb, _D), 1)
    inv_freq_b = jnp.exp((db // 2).astype(jnp.float32) * _NEG2LOG)
    parity = (db % 2).astype(jnp.float32)
    b = lax.broadcasted_iota(jnp.int32, (nb, _D), 0).astype(jnp.float32)
    small_ang = b * inv_freq_b + parity * (math.pi / 2.0)
    bb_ref[0] = jnp.sin(small_ang)
    bb_ref[1] = jnp.sin(small_ang + math.pi / 2.0)


def _make_sc_tables(sc_rows):
    na = sc_rows // _NB
    return pl.pallas_call(
        _tables_body,
        out_shape=(
            jax.ShapeDtypeStruct((2, na, _D), jnp.float32),
            jax.ShapeDtypeStruct((2, _NB, _D), jnp.float32),
        ),
    )()


def _sc_add(emb, pq, bb, nrows):
    napw = nrows // (16 * _NB)
    n_chunks = napw
    gstride = nrows // 16
    ring = min(4, n_chunks)

    mesh = plsc.VectorSubcoreMesh(core_axis_name="c", subcore_axis_name="s")

    @functools.partial(
        pl.kernel,
        out_type=jax.ShapeDtypeStruct((nrows, _D), jnp.float32),
        mesh=mesh,
        scratch_types=[
            pltpu.VMEM((2, napw, _D), jnp.float32),
            pltpu.VMEM((2, _CR, _D), jnp.float32),
            pltpu.VMEM((_CR, _D), jnp.float32),
            pltpu.VMEM((_CR, _D), jnp.float32),
            pltpu.VMEM((_CR, _D), jnp.float32),
            pltpu.VMEM((_CR, _D), jnp.float32),
            pltpu.VMEM((_CR, _D), jnp.float32),
            pltpu.VMEM((_CR, _D), jnp.float32),
            pltpu.SemaphoreType.DMA,
            pltpu.SemaphoreType.DMA,
            pltpu.SemaphoreType.DMA,
            pltpu.SemaphoreType.DMA,
            pltpu.SemaphoreType.DMA,
            pltpu.SemaphoreType.DMA,
            pltpu.SemaphoreType.DMA,
            pltpu.SemaphoreType.DMA,
        ],
    )
    def k(emb_hbm, pq_hbm, bb_hbm, out_hbm,
          pq_v, bb_v, in0, in1, in2, in3, out0, out1,
          isem0, isem1, isem2, isem3, osem0, osem1, tsem0, tsem1):
        cid = lax.axis_index("c")
        sid = lax.axis_index("s")
        wid = sid * 2 + cid
        g = wid // 2
        h = wid % 2
        base = g * gstride + h * _CR

        ins = (in0, in1, in2, in3)
        isems = (isem0, isem1, isem2, isem3)
        outs = (out0, out1)
        osems = (osem0, osem1)

        def in_copy(ci, buf, sem):
            return pltpu.make_async_copy(
                emb_hbm.at[pl.ds(base + ci * _NB, _CR)], buf, sem)

        def out_copy(ci, buf, sem):
            return pltpu.make_async_copy(
                buf, out_hbm.at[pl.ds(base + ci * _NB, _CR)], sem)

        for kk in range(ring):
            in_copy(kk, ins[kk], isems[kk]).start()
        tc_pq = pltpu.make_async_copy(
            pq_hbm.at[:, pl.ds(g * napw, napw), :], pq_v, tsem0)
        tc_bb = pltpu.make_async_copy(
            bb_hbm.at[:, pl.ds(h * _CR, _CR), :], bb_v, tsem1)
        tc_pq.start()
        tc_bb.start()
        tc_pq.wait()
        tc_bb.wait()

        for ci in range(n_chunks):
            in_b = ins[ci % 4]
            out_b = outs[ci % 2]
            osem = osems[ci % 2]
            in_copy(ci, in_b, isems[ci % 4]).wait()
            if ci >= 2:
                out_copy(ci - 2, out_b, osem).wait()

            @pl.loop(0, _NG)
            def _group(gg):
                sl = pl.ds(gg * 16, 16)
                p = pq_v[0, ci, sl]
                qv = pq_v[1, ci, sl]
                for r in range(_CR):
                    out_b[r, sl] = (in_b[r, sl]
                                    + p * bb_v[1, r, sl]
                                    + qv * bb_v[0, r, sl])

            out_copy(ci, out_b, osem).start()
            if ci + 4 < n_chunks:
                in_copy(ci + 4, in_b, isems[ci % 4]).start()

        for ci in range(max(0, n_chunks - 2), n_chunks):
            out_copy(ci, outs[ci % 2], osems[ci % 2]).wait()

    return k(emb, pq, bb)


_ROWS_PER_BLOCK = 512
_A_PER_BLOCK = _ROWS_PER_BLOCK // _NB
_SC_ROWS = 512


def _tc_full(emb, seq_len):
    nblk = seq_len // _ROWS_PER_BLOCK

    def body(emb_ref, o_ref, bb_ref):
        i = pl.program_id(0)

        @pl.when(i == 0)
        def _():
            db = lax.broadcasted_iota(jnp.int32, (_NB, _D), 1)
            inv_freq_b = jnp.exp((db // 2).astype(jnp.float32) * _NEG2LOG)
            parity = (db % 2).astype(jnp.float32)
            b = lax.broadcasted_iota(
                jnp.int32, (_NB, _D), 0).astype(jnp.float32)
            small_ang = b * inv_freq_b + parity * (math.pi / 2.0)
            bb_ref[0] = jnp.sin(small_ang)
            bb_ref[1] = jnp.sin(small_ang + math.pi / 2.0)

        d = lax.broadcasted_iota(jnp.int32, (_A_PER_BLOCK, _D), 1)
        inv_freq = jnp.exp((d // 2).astype(jnp.float32) * _NEG2LOG)
        al = lax.broadcasted_iota(
            jnp.int32, (_A_PER_BLOCK, _D), 0).astype(jnp.float32)
        a0 = (i * _A_PER_BLOCK).astype(jnp.float32)
        big_ang = ((a0 + al) * float(_NB)) * inv_freq
        p = jnp.sin(big_ang)[:, None, :]
        q = jnp.sin(big_ang + math.pi / 2.0)[:, None, :]
        sb = bb_ref[0][None, :, :]
        cb = bb_ref[1][None, :, :]
        emb3 = emb_ref[...].reshape(_A_PER_BLOCK, _NB, _D)
        out3 = emb3 + p * cb + q * sb
        o_ref[...] = out3.reshape(_ROWS_PER_BLOCK, _D)

    return pl.pallas_call(
        body,
        grid=(nblk,),
        in_specs=[
            pl.BlockSpec((_ROWS_PER_BLOCK, _D), lambda i: (i, 0)),
        ],
        out_specs=pl.BlockSpec((_ROWS_PER_BLOCK, _D), lambda i: (i, 0)),
        out_shape=jax.ShapeDtypeStruct((seq_len, _D), jnp.float32),
        scratch_shapes=[pltpu.VMEM((2, _NB, _D), jnp.float32)],
    )(emb)


def kernel(x, emb_table):
    seq_len = x.shape[1]
    tc_out = _tc_full(emb_table, seq_len)
    pq_sc, bb = _make_sc_tables(_SC_ROWS)
    sc_out = _sc_add(emb_table, pq_sc, bb, _SC_ROWS)
    out = lax.dynamic_update_slice(tc_out, sc_out, (0, 0))
    return out[None]
